# baseline (speedup 1.0000x reference)
import functools

import jax
import jax.numpy as jnp
from jax import lax
from jax.experimental import pallas as pl
from jax.experimental.pallas import tpu as pltpu
from jax.experimental.pallas import tpu_sc as plsc

N = 10000
E = 320000
D = 128

NC = 2
NS = 16
NW = NC * NS
L = 16
CHUNK = 128
CPW = 80
NCH = CPW * NW + 8
E_PAD = NCH * CHUNK
ROWS_PER_SUB = 640
R = ROWS_PER_SUB * NS
BLK = 400


def _mesh():
    return plsc.VectorSubcoreMesh(core_axis_name="c", subcore_axis_name="s")


def _zero_acc_slice(rows, acc, row0):
    def zrow(i, c):
        for j in range(D // L):
            rows[i, pl.ds(j * L, L)] = jnp.zeros((L,), jnp.float32)
        return c

    lax.fori_loop(0, CHUNK, zrow, 0)
    for blk in range(ROWS_PER_SUB // CHUNK):
        pltpu.sync_copy(rows, acc.at[pl.ds(row0 + blk * CHUNK, CHUNK)])


def _sc_agg(h, src2, dst2):
    @functools.partial(
        pl.kernel,
        out_type=jax.ShapeDtypeStruct((NC, R, D), jnp.float32),
        mesh=_mesh(),
        scratch_types=[
            pltpu.VMEM((CPW // 2 + 8, CHUNK), jnp.int32),
            pltpu.VMEM((CPW // 2, CHUNK), jnp.int32),
            pltpu.VMEM((CHUNK, D), jnp.float32),
            pltpu.VMEM((CHUNK, D), jnp.float32),
            pltpu.VMEM_SHARED((R, D), jnp.float32),
            pltpu.SemaphoreType.DMA,
            pltpu.SemaphoreType.DMA,
        ],
    )
    def k(x_hbm, src_hbm, dst_hbm, parts_hbm,
          sidx, didx, rows0, rows1, acc, sem0, sem1):
        cid = lax.axis_index("c")
        sid = lax.axis_index("s")
        wid = sid * NC + cid
        row0 = sid * ROWS_PER_SUB

        _zero_acc_slice(rows0, acc, row0)
        plsc.subcore_barrier()

        half_n = CPW // 2
        for half in range(2):
            hb = pl.multiple_of(wid * CPW + half * half_n, 8)
            pltpu.sync_copy(src_hbm.at[pl.ds(hb, half_n + 8)], sidx)
            pltpu.sync_copy(dst_hbm.at[pl.ds(hb, half_n)], didx)

            pltpu.async_copy(x_hbm.at[sidx.at[0]], rows0, sem0)
            pltpu.async_copy(x_hbm.at[sidx.at[1]], rows1, sem1)

            def body(i, carry):
                c = i * 2
                pltpu.make_async_copy(x_hbm.at[pl.ds(0, CHUNK)], rows0, sem0).wait()
                pltpu.sync_copy(rows0, acc.at[didx.at[c]], add=True)
                pltpu.async_copy(x_hbm.at[sidx.at[c + 2]], rows0, sem0)
                pltpu.make_async_copy(x_hbm.at[pl.ds(0, CHUNK)], rows1, sem1).wait()
                pltpu.sync_copy(rows1, acc.at[didx.at[c + 1]], add=True)
                pltpu.async_copy(x_hbm.at[sidx.at[c + 3]], rows1, sem1)
                return carry

            lax.fori_loop(0, half_n // 2, body, 0)
            pltpu.make_async_copy(x_hbm.at[pl.ds(0, CHUNK)], rows0, sem0).wait()
            pltpu.make_async_copy(x_hbm.at[pl.ds(0, CHUNK)], rows1, sem1).wait()

        plsc.subcore_barrier()
        pltpu.sync_copy(acc.at[pl.ds(row0, ROWS_PER_SUB)],
                        parts_hbm.at[cid, pl.ds(row0, ROWS_PER_SUB)])

    return k(h, src2, dst2)


def _sc_deg(dst2):
    @functools.partial(
        pl.kernel,
        out_type=jax.ShapeDtypeStruct((NC, R, D), jnp.float32),
        mesh=_mesh(),
        scratch_types=[
            pltpu.VMEM((CPW, CHUNK), jnp.int32),
            pltpu.VMEM((CHUNK, D), jnp.float32),
            pltpu.VMEM_SHARED((R, D), jnp.float32),
        ],
    )
    def k(dst_hbm, degp_hbm, didx, rows, acc):
        cid = lax.axis_index("c")
        sid = lax.axis_index("s")
        wid = sid * NC + cid
        row0 = sid * ROWS_PER_SUB

        _zero_acc_slice(rows, acc, row0)

        def orow(i, c):
            for j in range(D // L):
                rows[i, pl.ds(j * L, L)] = jnp.ones((L,), jnp.float32)
            return c

        lax.fori_loop(0, CHUNK, orow, 0)
        plsc.subcore_barrier()

        ch0 = pl.multiple_of(wid * CPW, CPW)
        pltpu.sync_copy(dst_hbm.at[pl.ds(ch0, CPW)], didx)

        def body(c, carry):
            pltpu.sync_copy(rows, acc.at[didx.at[c]], add=True)
            return carry

        lax.fori_loop(0, CPW, body, 0)
        plsc.subcore_barrier()
        pltpu.sync_copy(acc.at[pl.ds(row0, ROWS_PER_SUB)],
                        degp_hbm.at[cid, pl.ds(row0, ROWS_PER_SUB)])

    return k(dst2)


def _tc_layer(parts, degp, h, WlT, bl, WrT, last):
    def body(p_ref, d_ref, h_ref, wl_ref, b_ref, wr_ref, o_ref):
        p = p_ref[0] + p_ref[1]
        d = d_ref[0, :, 0:1] + d_ref[1, :, 0:1]
        mean = p / jnp.maximum(d, 1.0)
        o = (jnp.dot(mean, wl_ref[...], preferred_element_type=jnp.float32)
             + b_ref[...]
             + jnp.dot(h_ref[...], wr_ref[...], preferred_element_type=jnp.float32))
        if last:
            m = jnp.max(o, axis=1, keepdims=True)
            e = jnp.exp(o - m)
            s = jnp.sum(e, axis=1, keepdims=True)
            o_ref[...] = (o - m) - jnp.log(s)
        else:
            o_ref[...] = jnp.maximum(o, 0.0)

    return pl.pallas_call(
        body,
        grid=(N // BLK,),
        in_specs=[
            pl.BlockSpec((NC, BLK, D), lambda i: (0, i, 0)),
            pl.BlockSpec((NC, BLK, D), lambda i: (0, i, 0)),
            pl.BlockSpec((BLK, D), lambda i: (i, 0)),
            pl.BlockSpec((D, D), lambda i: (0, 0)),
            pl.BlockSpec((1, D), lambda i: (0, 0)),
            pl.BlockSpec((D, D), lambda i: (0, 0)),
        ],
        out_specs=pl.BlockSpec((BLK, D), lambda i: (i, 0)),
        out_shape=jax.ShapeDtypeStruct((N, D), jnp.float32),
    )(parts, degp, h, WlT, bl, WrT)


def kernel(x, edge_index, W1l, b1l, W1r, W2l, b2l, W2r):
    src = edge_index[0].astype(jnp.int32)
    dst = edge_index[1].astype(jnp.int32)
    pad = E_PAD - E
    src2 = jnp.concatenate([src, jnp.zeros((pad,), jnp.int32)]).reshape(NCH, CHUNK)
    dst2 = jnp.concatenate([dst, jnp.full((pad,), N, jnp.int32)]).reshape(NCH, CHUNK)

    degp = _sc_deg(dst2)
    parts1 = _sc_agg(x, src2, dst2)
    h = _tc_layer(parts1, degp, x, W1l.T, b1l.reshape(1, D), W1r.T, last=False)
    parts2 = _sc_agg(h, src2, dst2)
    out = _tc_layer(parts2, degp, h, W2l.T, b2l.reshape(1, D), W2r.T, last=True)
    return out
